# baseline (speedup 1.0000x reference)
import functools

import jax
import jax.numpy as jnp
from jax.experimental import pallas as pl
from jax.experimental.pallas import tpu as pltpu


def _moe_block_kernel(x_ref, gw_ref, gb_ref, w_hbm_ref, b_ref, o_ref,
                      stag_ref, sem_ref, *, n_exp):
    def start_copy(j):
        pltpu.make_async_copy(
            w_hbm_ref.at[j], stag_ref.at[j % 2], sem_ref.at[j % 2]
        ).start()

    start_copy(0)

    xb = x_ref[...].astype(jnp.bfloat16)

    logits = jax.lax.dot_general(
        gw_ref[...], xb, (((1,), (1,)), ((), ())),
        preferred_element_type=jnp.float32,
    ) + gb_ref[...]
    p = jax.nn.softmax(logits, axis=0)
    m1 = jnp.max(p, axis=0, keepdims=True)
    row = jax.lax.broadcasted_iota(jnp.int32, p.shape, 0)
    first = jnp.min(jnp.where(p == m1, row, p.shape[0]), axis=0, keepdims=True)
    m2 = jnp.max(jnp.where(row == first, -1.0, p), axis=0, keepdims=True)
    wsum = (m1 + m2).T

    acc = None
    for j in range(n_exp):
        if j + 1 < n_exp:
            start_copy(j + 1)
        pltpu.make_async_copy(
            w_hbm_ref.at[j], stag_ref.at[j % 2], sem_ref.at[j % 2]
        ).wait()
        y = jax.lax.dot_general(
            xb, stag_ref[j % 2].astype(jnp.bfloat16), (((1,), (1,)), ((), ())),
            preferred_element_type=jnp.float32,
        )
        y = jnp.maximum(y + b_ref[j], 0.0)
        acc = y if acc is None else acc + y
    o_ref[...] = acc * wsum


def _moe_pallas(xf, gw, gb2, ew, eb3, *, blk):
    T, D = xf.shape
    E = gw.shape[0]
    n_tblk = T // blk
    return pl.pallas_call(
        functools.partial(_moe_block_kernel, n_exp=E),
        grid=(n_tblk,),
        in_specs=[
            pl.BlockSpec((blk, D), lambda t: (t, 0)),
            pl.BlockSpec((E, D), lambda t: (0, 0)),
            pl.BlockSpec((E, 1), lambda t: (0, 0)),
            pl.BlockSpec(memory_space=pltpu.MemorySpace.HBM),
            pl.BlockSpec((E, 1, D), lambda t: (0, 0, 0)),
        ],
        out_specs=pl.BlockSpec((blk, D), lambda t: (t, 0)),
        out_shape=jax.ShapeDtypeStruct((T, D), jnp.float32),
        scratch_shapes=[
            pltpu.VMEM((2, D, D), jnp.float32),
            pltpu.SemaphoreType.DMA((2,)),
        ],
        compiler_params=pltpu.CompilerParams(
            dimension_semantics=("arbitrary",)
        ),
    )(xf, gw, gb2, ew, eb3)


def kernel(x, gate_W, gate_b, expert_W, expert_b):
    B, S, D = x.shape
    E = gate_W.shape[0]
    T = B * S

    xf = x.reshape(T, D)
    gw = gate_W.astype(jnp.bfloat16)
    gb2 = gate_b.reshape(E, 1)
    eb3 = expert_b.reshape(E, 1, D)

    out = _moe_pallas(xf, gw, gb2, expert_W, eb3, blk=1024)
    return out.reshape(B, S, D)


# speedup vs baseline: 1.1173x; 1.1173x over previous
import functools

import jax
import jax.numpy as jnp
from jax.experimental import pallas as pl
from jax.experimental.pallas import tpu as pltpu


def _moe_block_kernel(x_ref, gw_ref, gb_ref, w_ref, b_ref, o_ref, *, n_exp):
    xb = x_ref[...].astype(jnp.bfloat16)

    logits = jax.lax.dot_general(
        gw_ref[...], xb, (((1,), (1,)), ((), ())),
        preferred_element_type=jnp.float32,
    ) + gb_ref[...]
    p = jax.nn.softmax(logits, axis=0)
    m1 = jnp.max(p, axis=0, keepdims=True)
    row = jax.lax.broadcasted_iota(jnp.int32, p.shape, 0)
    first = jnp.min(jnp.where(p == m1, row, p.shape[0]), axis=0, keepdims=True)
    m2 = jnp.max(jnp.where(row == first, -1.0, p), axis=0, keepdims=True)
    wsum = (m1 + m2).T

    acc = None
    for j in range(n_exp):
        y = jax.lax.dot_general(
            xb, w_ref[j].astype(jnp.bfloat16), (((1,), (1,)), ((), ())),
            preferred_element_type=jnp.float32,
        )
        y = jnp.maximum(y + b_ref[j], 0.0)
        acc = y if acc is None else acc + y
    o_ref[...] = acc * wsum


def _moe_pallas(xf, gw, gb2, ew, eb3, *, blk):
    T, D = xf.shape
    E = gw.shape[0]
    n_tblk = T // blk
    return pl.pallas_call(
        functools.partial(_moe_block_kernel, n_exp=E),
        grid=(n_tblk,),
        in_specs=[
            pl.BlockSpec((blk, D), lambda t: (t, 0)),
            pl.BlockSpec((E, D), lambda t: (0, 0)),
            pl.BlockSpec((E, 1), lambda t: (0, 0)),
            pl.BlockSpec((E, D, D), lambda t: (0, 0, 0)),
            pl.BlockSpec((E, 1, D), lambda t: (0, 0, 0)),
        ],
        out_specs=pl.BlockSpec((blk, D), lambda t: (t, 0)),
        out_shape=jax.ShapeDtypeStruct((T, D), jnp.float32),
        compiler_params=pltpu.CompilerParams(
            dimension_semantics=("arbitrary",)
        ),
    )(xf, gw, gb2, ew, eb3)


def kernel(x, gate_W, gate_b, expert_W, expert_b):
    B, S, D = x.shape
    E = gate_W.shape[0]
    T = B * S

    xf = x.reshape(T, D)
    gw = gate_W.astype(jnp.bfloat16)
    gb2 = gate_b.reshape(E, 1)
    eb3 = expert_b.reshape(E, 1, D)

    out = _moe_pallas(xf, gw, gb2, expert_W, eb3, blk=1024)
    return out.reshape(B, S, D)


# speedup vs baseline: 1.1323x; 1.0134x over previous
import functools

import jax
import jax.numpy as jnp
from jax.experimental import pallas as pl
from jax.experimental.pallas import tpu as pltpu


def _moe_block_kernel(x_ref, gw_ref, gb_ref, w_ref, b_ref, o_ref, *, n_exp):
    xb = x_ref[...]

    logits = jax.lax.dot_general(
        gw_ref[...], xb, (((1,), (1,)), ((), ())),
        preferred_element_type=jnp.float32,
    ) + gb_ref[...]
    p = jax.nn.softmax(logits, axis=0)
    m1 = jnp.max(p, axis=0, keepdims=True)
    row = jax.lax.broadcasted_iota(jnp.int32, p.shape, 0)
    first = jnp.min(jnp.where(p == m1, row, p.shape[0]), axis=0, keepdims=True)
    m2 = jnp.max(jnp.where(row == first, -1.0, p), axis=0, keepdims=True)
    wsum = (m1 + m2).T

    acc = None
    for j in range(n_exp):
        y = jax.lax.dot_general(
            xb, w_ref[j], (((1,), (1,)), ((), ())),
            preferred_element_type=jnp.float32,
        )
        y = jnp.maximum(y + b_ref[j], 0.0)
        acc = y if acc is None else acc + y
    o_ref[...] = acc * wsum


def _moe_pallas(xf, gw, gb2, ew, eb3, *, blk):
    T, D = xf.shape
    E = gw.shape[0]
    n_tblk = T // blk
    return pl.pallas_call(
        functools.partial(_moe_block_kernel, n_exp=E),
        grid=(n_tblk,),
        in_specs=[
            pl.BlockSpec((blk, D), lambda t: (t, 0)),
            pl.BlockSpec((E, D), lambda t: (0, 0)),
            pl.BlockSpec((E, 1), lambda t: (0, 0)),
            pl.BlockSpec((E, D, D), lambda t: (0, 0, 0)),
            pl.BlockSpec((E, 1, D), lambda t: (0, 0, 0)),
        ],
        out_specs=pl.BlockSpec((blk, D), lambda t: (t, 0)),
        out_shape=jax.ShapeDtypeStruct((T, D), jnp.float32),
        compiler_params=pltpu.CompilerParams(
            dimension_semantics=("arbitrary",)
        ),
    )(xf, gw, gb2, ew, eb3)


def kernel(x, gate_W, gate_b, expert_W, expert_b):
    B, S, D = x.shape
    E = gate_W.shape[0]
    T = B * S

    xf = x.reshape(T, D)
    gw = gate_W
    gb2 = gate_b.reshape(E, 1)
    eb3 = expert_b.reshape(E, 1, D)

    out = _moe_pallas(xf, gw, gb2, expert_W, eb3, blk=1024)
    return out.reshape(B, S, D)
